# Initial kernel scaffold; baseline (speedup 1.0000x reference)
import jax
import jax.numpy as jnp
from jax.experimental import pallas as pl


def kernel(input, edge_index, edge_weight_road, edge_r, W, Wr, a):
    raise NotImplementedError("write your pallas kernel here")


# speedup vs baseline: 3.3443x; 3.3443x over previous
import functools

import jax
import jax.numpy as jnp
from jax import lax
from jax.experimental import pallas as pl
from jax.experimental.pallas import tpu as pltpu
from jax.experimental.pallas import tpu_sc as plsc

N = 10000
E = 320000
D = 128
ALPHA = 0.2

NC, NS, L = 2, 16, 16
NW = NC * NS
K = 80
CPW = (E // NW) // K
NPAD = 10240
RPT = NPAD // NS
ZR = 128
NH = 80


def _dense_node_body(x_ref, w_ref, a_ref, h_ref, s1_ref, s2_ref):
    h = jnp.dot(x_ref[...], w_ref[...], preferred_element_type=jnp.float32)
    h_ref[...] = h
    a1 = a_ref[:, 0:D]
    a2 = a_ref[:, D:2 * D]
    s1_ref[...] = lax.dot_general(h, a1, (((1,), (1,)), ((), ())),
                                  preferred_element_type=jnp.float32)
    s2_ref[...] = lax.dot_general(h, a2, (((1,), (1,)), ((), ())),
                                  preferred_element_type=jnp.float32)


def _dense_node(x, W, a):
    NB = 2000
    return pl.pallas_call(
        _dense_node_body,
        grid=(N // NB,),
        in_specs=[
            pl.BlockSpec((NB, D), lambda i: (i, 0)),
            pl.BlockSpec((D, D), lambda i: (0, 0)),
            pl.BlockSpec((1, 3 * D + 1), lambda i: (0, 0)),
        ],
        out_specs=[
            pl.BlockSpec((NB, D), lambda i: (i, 0)),
            pl.BlockSpec((NB, 1), lambda i: (i, 0)),
            pl.BlockSpec((NB, 1), lambda i: (i, 0)),
        ],
        out_shape=[
            jax.ShapeDtypeStruct((N, D), jnp.float32),
            jax.ShapeDtypeStruct((N, 1), jnp.float32),
            jax.ShapeDtypeStruct((N, 1), jnp.float32),
        ],
    )(x, W, a)


def _dense_edge_body(er_ref, ew_ref, wr_ref, a_ref, t_ref):
    a3 = a_ref[:, 2 * D:3 * D]
    wra3 = lax.dot_general(wr_ref[...], a3, (((1,), (1,)), ((), ())),
                           preferred_element_type=jnp.float32)
    a4 = a_ref[0, 3 * D]
    t_ref[...] = (jnp.dot(er_ref[...], wra3, preferred_element_type=jnp.float32)
                  + a4 * ew_ref[...])


def _dense_edge(edge_r, ew, Wr, a):
    EB = 16000
    dr = edge_r.shape[1]
    return pl.pallas_call(
        _dense_edge_body,
        grid=(E // EB,),
        in_specs=[
            pl.BlockSpec((EB, dr), lambda i: (i, 0)),
            pl.BlockSpec((EB, 1), lambda i: (i, 0)),
            pl.BlockSpec((dr, D), lambda i: (0, 0)),
            pl.BlockSpec((1, 3 * D + 1), lambda i: (0, 0)),
        ],
        out_specs=pl.BlockSpec((EB, 1), lambda i: (i, 0)),
        out_shape=jax.ShapeDtypeStruct((E, 1), jnp.float32),
    )(edge_r, ew, Wr, a)


def _sc_edge_body(h_hbm, s1_hbm, s2_hbm, src_hbm, dst_hbm, t_hbm, hp_out,
                  rs_out, s1_v, s2_v, src_v, dst_v, t_v, rows_in, rs_local,
                  idx_id, hp_acc, rs_shared, sem):
    cid = lax.axis_index("c")
    sid = lax.axis_index("s")
    wid = cid * NS + sid

    pltpu.sync_copy(s1_hbm, s1_v)
    pltpu.sync_copy(s2_hbm, s2_v)

    zvec = jnp.zeros((L,), jnp.float32)

    def _zero_rows(i, _):
        for j in range(D // L):
            rs_local[i, pl.ds(j * L, L)] = zvec
            rows_in[i, pl.ds(j * L, L)] = zvec
        return ()

    lax.fori_loop(0, NH, _zero_rows, ())
    for kblk in range(RPT // K):
        pltpu.sync_copy(rows_in, hp_acc.at[pl.ds(sid * RPT + kblk * K, K)])

    @pl.when(sid == 0)
    def _zero_rs_shared():
        pltpu.sync_copy(rows_in, rs_shared)

    for g in range(NH // L):
        idx_id[pl.ds(g * L, L)] = lax.iota(jnp.int32, L) + g * L
    plsc.subcore_barrier()

    lanes = lax.iota(jnp.int32, L)
    lane_masks = [lanes == i for i in range(L)]
    bcast_dn = lax.GatherDimensionNumbers(
        offset_dims=(), collapsed_slice_dims=(0,), start_index_map=(0,))

    def _bcast_lane(vec, i):
        idx = jnp.full((L, 1), i, dtype=jnp.int32)
        return lax.gather(vec, idx, bcast_dn, (1,),
                          mode=lax.GatherScatterMode.PROMISE_IN_BOUNDS)

    def _chunk(c, _):
        j = wid * CPW + c
        pltpu.sync_copy(src_hbm.at[j], src_v)
        pltpu.sync_copy(dst_hbm.at[j], dst_v)
        pltpu.sync_copy(t_hbm.at[j], t_v)
        pltpu.async_copy(h_hbm.at[dst_v.at[0]], rows_in, sem).wait()
        for g in range(K // L):
            sl = pl.ds(g * L, L)
            si = src_v[0, sl]
            di = dst_v[0, sl]
            score = (plsc.load_gather(s1_v, [si])
                     + plsc.load_gather(s2_v, [di]) + t_v[0, sl])
            e = jnp.exp(-jnp.maximum(score, ALPHA * score))
            hi = lax.shift_right_logical(si, 7)
            lo = lax.bitwise_and(si, 127)
            for i in range(L):
                row = g * L + i
                plsc.addupdate_scatter(rs_local, [hi, lo], e,
                                       mask=lane_masks[i])
                eb = _bcast_lane(e, i)
                for f in range(D // L):
                    fs = pl.ds(f * L, L)
                    rows_in[row, fs] = rows_in[row, fs] * eb
        pltpu.sync_copy(rows_in, hp_acc.at[src_v.at[0]], add=True)
        return ()

    lax.fori_loop(0, CPW, _chunk, ())

    plsc.subcore_barrier()
    pltpu.sync_copy(rs_local, rs_shared.at[idx_id], add=True)
    plsc.subcore_barrier()

    @pl.when(sid == 0)
    def _dump_rs():
        pltpu.sync_copy(rs_shared, rs_out.at[cid])

    for kblk in range(RPT // ZR):
        r0 = sid * RPT + kblk * ZR
        pltpu.sync_copy(hp_acc.at[pl.ds(r0, ZR)], hp_out.at[cid, pl.ds(r0, ZR)])
    plsc.subcore_barrier()


_sc_edge = functools.partial(
    pl.kernel,
    out_type=[
        jax.ShapeDtypeStruct((NC, NPAD, D), jnp.float32),
        jax.ShapeDtypeStruct((NC, NH, D), jnp.float32),
    ],
    mesh=plsc.VectorSubcoreMesh(core_axis_name="c", subcore_axis_name="s",
                                num_cores=NC, num_subcores=NS),
    scratch_types=[
        pltpu.VMEM((N,), jnp.float32),
        pltpu.VMEM((N,), jnp.float32),
        pltpu.VMEM((1, K), jnp.int32),
        pltpu.VMEM((1, K), jnp.int32),
        pltpu.VMEM((1, K), jnp.float32),
        pltpu.VMEM((K, D), jnp.float32),
        pltpu.VMEM((NH, D), jnp.float32),
        pltpu.VMEM((NH,), jnp.int32),
        pltpu.VMEM_SHARED((NPAD, D), jnp.float32),
        pltpu.VMEM_SHARED((NH, D), jnp.float32),
        pltpu.SemaphoreType.DMA,
    ],
    compiler_params=pltpu.CompilerParams(needs_layout_passes=False),
)(_sc_edge_body)


def _finalize_body(hp_ref, rs_ref, o_ref):
    top = hp_ref[0, :, :] + hp_ref[1, :, :]
    rs = rs_ref[0, :, :] + rs_ref[1, :, :]
    x = top / (rs + 9e-15)
    o_ref[...] = jnp.where(x > 0, x, jnp.exp(jnp.minimum(x, 0.0)) - 1.0)


def _finalize(hp, rs):
    NB = 2000
    return pl.pallas_call(
        _finalize_body,
        grid=(N // NB,),
        in_specs=[
            pl.BlockSpec((NC, NB, D), lambda i: (0, i, 0)),
            pl.BlockSpec((NC, NB, 1), lambda i: (0, i, 0)),
        ],
        out_specs=pl.BlockSpec((NB, D), lambda i: (i, 0)),
        out_shape=jax.ShapeDtypeStruct((N, D), jnp.float32),
    )(hp, rs)


def kernel(input, edge_index, edge_weight_road, edge_r, W, Wr, a):
    h, s1, s2 = _dense_node(input, W, a)
    t = _dense_edge(edge_r, edge_weight_road.reshape(E, 1), Wr, a)
    src = edge_index[0].astype(jnp.int32).reshape(NW * CPW, 1, K)
    dst = edge_index[1].astype(jnp.int32).reshape(NW * CPW, 1, K)
    hp, rs = _sc_edge(h, s1.reshape(N), s2.reshape(N), src, dst,
                      t.reshape(NW * CPW, 1, K))
    return _finalize(hp, rs.reshape(NC, NH * D)[:, :N].reshape(NC, N, 1))


# speedup vs baseline: 5.6406x; 1.6866x over previous
import functools

import jax
import jax.numpy as jnp
from jax import lax
from jax.experimental import pallas as pl
from jax.experimental.pallas import tpu as pltpu
from jax.experimental.pallas import tpu_sc as plsc

N = 10000
E = 320000
D = 128
ALPHA = 0.2

NC, NS, L = 2, 16, 16
NW = NC * NS
K = 80
CPW = (E // NW) // K
NPAD = 10240
RPT = NPAD // NS
ZR = 128
NH = 80


def _dense_node_body(x_ref, w_ref, a_ref, h_ref, s1_ref, s2_ref):
    h = jnp.dot(x_ref[...], w_ref[...], preferred_element_type=jnp.float32)
    h_ref[...] = h
    a1 = a_ref[:, 0:D]
    a2 = a_ref[:, D:2 * D]
    s1_ref[...] = lax.dot_general(h, a1, (((1,), (1,)), ((), ())),
                                  preferred_element_type=jnp.float32)
    s2_ref[...] = lax.dot_general(h, a2, (((1,), (1,)), ((), ())),
                                  preferred_element_type=jnp.float32)


def _dense_node(x, W, a):
    NB = 2000
    return pl.pallas_call(
        _dense_node_body,
        grid=(N // NB,),
        in_specs=[
            pl.BlockSpec((NB, D), lambda i: (i, 0)),
            pl.BlockSpec((D, D), lambda i: (0, 0)),
            pl.BlockSpec((1, 3 * D + 1), lambda i: (0, 0)),
        ],
        out_specs=[
            pl.BlockSpec((NB, D), lambda i: (i, 0)),
            pl.BlockSpec((NB, 1), lambda i: (i, 0)),
            pl.BlockSpec((NB, 1), lambda i: (i, 0)),
        ],
        out_shape=[
            jax.ShapeDtypeStruct((N, D), jnp.float32),
            jax.ShapeDtypeStruct((N, 1), jnp.float32),
            jax.ShapeDtypeStruct((N, 1), jnp.float32),
        ],
    )(x, W, a)


ER = E // D


def _dense_edge_body(er_ref, ew_ref, wr_ref, a_ref, t_ref):
    a3 = a_ref[:, 2 * D:3 * D]
    wra3 = lax.dot_general(wr_ref[...], a3, (((1,), (1,)), ((), ())),
                           preferred_element_type=jnp.float32)
    a4 = a_ref[0, 3 * D]
    acc = a4 * ew_ref[...]
    for j in range(8):
        acc = acc + wra3[j, 0] * er_ref[j, :, :]
    t_ref[...] = acc


def _dense_edge(edge_r_t, ew, Wr, a):
    RB = ER
    return pl.pallas_call(
        _dense_edge_body,
        grid=(ER // RB,),
        in_specs=[
            pl.BlockSpec((8, RB, D), lambda i: (0, i, 0)),
            pl.BlockSpec((RB, D), lambda i: (i, 0)),
            pl.BlockSpec((8, D), lambda i: (0, 0)),
            pl.BlockSpec((1, 3 * D + 1), lambda i: (0, 0)),
        ],
        out_specs=pl.BlockSpec((RB, D), lambda i: (i, 0)),
        out_shape=jax.ShapeDtypeStruct((ER, D), jnp.float32),
    )(edge_r_t, ew, Wr, a)


def _sc_edge_body(h_hbm, s1_hbm, s2_hbm, src_hbm, dst_hbm, t_hbm, hp_out,
                  rs_out, s1_v, s2_v, src_v, dst_v, t_v, rows_in, rs_local,
                  idx_id, hp_acc, rs_shared, sem):
    cid = lax.axis_index("c")
    sid = lax.axis_index("s")
    wid = cid * NS + sid

    pltpu.sync_copy(s1_hbm, s1_v)
    pltpu.sync_copy(s2_hbm, s2_v)

    zvec = jnp.zeros((L,), jnp.float32)

    def _zero_rows(i, _):
        for j in range(D // L):
            rs_local[i, pl.ds(j * L, L)] = zvec
            rows_in[i, pl.ds(j * L, L)] = zvec
        return ()

    lax.fori_loop(0, NH, _zero_rows, ())
    for kblk in range(RPT // K):
        pltpu.sync_copy(rows_in, hp_acc.at[pl.ds(sid * RPT + kblk * K, K)])

    @pl.when(sid == 0)
    def _zero_rs_shared():
        pltpu.sync_copy(rows_in, rs_shared)

    for g in range(NH // L):
        idx_id[pl.ds(g * L, L)] = lax.iota(jnp.int32, L) + g * L
    plsc.subcore_barrier()

    lanes = lax.iota(jnp.int32, L)
    lane_masks = [lanes == i for i in range(L)]
    bcast_dn = lax.GatherDimensionNumbers(
        offset_dims=(), collapsed_slice_dims=(0,), start_index_map=(0,))

    def _bcast_lane(vec, i):
        idx = jnp.full((L, 1), i, dtype=jnp.int32)
        return lax.gather(vec, idx, bcast_dn, (1,),
                          mode=lax.GatherScatterMode.PROMISE_IN_BOUNDS)

    def _chunk(c, _):
        e0 = (wid * CPW + c) * K
        pltpu.sync_copy(src_hbm.at[pl.ds(e0, K)], src_v)
        pltpu.sync_copy(dst_hbm.at[pl.ds(e0, K)], dst_v)
        pltpu.sync_copy(t_hbm.at[pl.ds(e0, K)], t_v)
        pltpu.async_copy(h_hbm.at[dst_v], rows_in, sem).wait()
        for g in range(K // L):
            sl = pl.ds(g * L, L)
            si = src_v[sl]
            di = dst_v[sl]
            score = (plsc.load_gather(s1_v, [si])
                     + plsc.load_gather(s2_v, [di]) + t_v[sl])
            e = jnp.exp(-jnp.maximum(score, ALPHA * score))
            hi = lax.shift_right_logical(si, 7)
            lo = lax.bitwise_and(si, 127)
            for i in range(L):
                row = g * L + i
                plsc.addupdate_scatter(rs_local, [hi, lo], e,
                                       mask=lane_masks[i])
                eb = _bcast_lane(e, i)
                for f in range(D // L):
                    fs = pl.ds(f * L, L)
                    rows_in[row, fs] = rows_in[row, fs] * eb
        pltpu.sync_copy(rows_in, hp_acc.at[src_v], add=True)
        return ()

    lax.fori_loop(0, CPW, _chunk, ())

    plsc.subcore_barrier()
    pltpu.sync_copy(rs_local, rs_shared.at[idx_id], add=True)
    plsc.subcore_barrier()

    @pl.when(sid == 0)
    def _dump_rs():
        pltpu.sync_copy(rs_shared, rs_out.at[cid])

    for kblk in range(RPT // ZR):
        r0 = sid * RPT + kblk * ZR
        pltpu.sync_copy(hp_acc.at[pl.ds(r0, ZR)], hp_out.at[cid, pl.ds(r0, ZR)])
    plsc.subcore_barrier()


_sc_edge = functools.partial(
    pl.kernel,
    out_type=[
        jax.ShapeDtypeStruct((NC, NPAD, D), jnp.float32),
        jax.ShapeDtypeStruct((NC, NH, D), jnp.float32),
    ],
    mesh=plsc.VectorSubcoreMesh(core_axis_name="c", subcore_axis_name="s",
                                num_cores=NC, num_subcores=NS),
    scratch_types=[
        pltpu.VMEM((N,), jnp.float32),
        pltpu.VMEM((N,), jnp.float32),
        pltpu.VMEM((K,), jnp.int32),
        pltpu.VMEM((K,), jnp.int32),
        pltpu.VMEM((K,), jnp.float32),
        pltpu.VMEM((K, D), jnp.float32),
        pltpu.VMEM((NH, D), jnp.float32),
        pltpu.VMEM((NH,), jnp.int32),
        pltpu.VMEM_SHARED((NPAD, D), jnp.float32),
        pltpu.VMEM_SHARED((NH, D), jnp.float32),
        pltpu.SemaphoreType.DMA,
    ],
    compiler_params=pltpu.CompilerParams(needs_layout_passes=False),
)(_sc_edge_body)


def _finalize_body(hp_ref, rs_ref, o_ref):
    top = hp_ref[0, :, :] + hp_ref[1, :, :]
    rs = rs_ref[0, :, :] + rs_ref[1, :, :]
    x = top / (rs + 9e-15)
    o_ref[...] = jnp.where(x > 0, x, jnp.exp(jnp.minimum(x, 0.0)) - 1.0)


def _finalize(hp, rs):
    NB = 2000
    return pl.pallas_call(
        _finalize_body,
        grid=(N // NB,),
        in_specs=[
            pl.BlockSpec((NC, NB, D), lambda i: (0, i, 0)),
            pl.BlockSpec((NC, NB, 1), lambda i: (0, i, 0)),
        ],
        out_specs=pl.BlockSpec((NB, D), lambda i: (i, 0)),
        out_shape=jax.ShapeDtypeStruct((N, D), jnp.float32),
    )(hp, rs)


def kernel(input, edge_index, edge_weight_road, edge_r, W, Wr, a):
    h, s1, s2 = _dense_node(input, W, a)
    t = _dense_edge(edge_r.T.reshape(8, ER, D),
                    edge_weight_road.reshape(ER, D), Wr, a)
    src = edge_index[0].astype(jnp.int32)
    dst = edge_index[1].astype(jnp.int32)
    hp, rs = _sc_edge(h, s1.reshape(N), s2.reshape(N), src, dst, t.reshape(E))
    return _finalize(hp, rs.reshape(NC, NH * D)[:, :N].reshape(NC, N, 1))


# speedup vs baseline: 8.2995x; 1.4714x over previous
import functools

import jax
import jax.numpy as jnp
from jax import lax
from jax.experimental import pallas as pl
from jax.experimental.pallas import tpu as pltpu
from jax.experimental.pallas import tpu_sc as plsc

N = 10000
E = 320000
D = 128
ALPHA = 0.2

NC, NS, L = 2, 16, 16
NW = NC * NS
K = 80
CPW = (E // NW) // K
NPAD = 10240
RPT = NPAD // NS
ZR = 128
NH = 80


def _dense_node_body(x_ref, w_ref, a_ref, h_ref, s1_ref, s2_ref):
    h = jnp.dot(x_ref[...], w_ref[...], preferred_element_type=jnp.float32)
    h_ref[...] = h
    a1 = a_ref[:, 0:D]
    a2 = a_ref[:, D:2 * D]
    s1_ref[...] = lax.dot_general(h, a1, (((1,), (1,)), ((), ())),
                                  preferred_element_type=jnp.float32)
    s2_ref[...] = lax.dot_general(h, a2, (((1,), (1,)), ((), ())),
                                  preferred_element_type=jnp.float32)


def _dense_node(x, W, a):
    NB = 2000
    return pl.pallas_call(
        _dense_node_body,
        grid=(N // NB,),
        in_specs=[
            pl.BlockSpec((NB, D), lambda i: (i, 0)),
            pl.BlockSpec((D, D), lambda i: (0, 0)),
            pl.BlockSpec((1, 3 * D + 1), lambda i: (0, 0)),
        ],
        out_specs=[
            pl.BlockSpec((NB, D), lambda i: (i, 0)),
            pl.BlockSpec((NB, 1), lambda i: (i, 0)),
            pl.BlockSpec((NB, 1), lambda i: (i, 0)),
        ],
        out_shape=[
            jax.ShapeDtypeStruct((N, D), jnp.float32),
            jax.ShapeDtypeStruct((N, 1), jnp.float32),
            jax.ShapeDtypeStruct((N, 1), jnp.float32),
        ],
    )(x, W, a)


ER = E // D


def _dense_edge_body(er_ref, ew_ref, wr_ref, a_ref, t_ref):
    a3 = a_ref[:, 2 * D:3 * D]
    wra3 = lax.dot_general(wr_ref[...], a3, (((1,), (1,)), ((), ())),
                           preferred_element_type=jnp.float32)
    a4 = a_ref[0, 3 * D]
    acc = a4 * ew_ref[...]
    for j in range(8):
        acc = acc + wra3[j, 0] * er_ref[j, :, :]
    t_ref[...] = acc


def _dense_edge(edge_r_t, ew, Wr, a):
    RB = ER
    return pl.pallas_call(
        _dense_edge_body,
        grid=(ER // RB,),
        in_specs=[
            pl.BlockSpec((8, RB, D), lambda i: (0, i, 0)),
            pl.BlockSpec((RB, D), lambda i: (i, 0)),
            pl.BlockSpec((8, D), lambda i: (0, 0)),
            pl.BlockSpec((1, 3 * D + 1), lambda i: (0, 0)),
        ],
        out_specs=pl.BlockSpec((RB, D), lambda i: (i, 0)),
        out_shape=jax.ShapeDtypeStruct((ER, D), jnp.float32),
    )(edge_r_t, ew, Wr, a)


def _sc_edge_body(h_hbm, s1_hbm, s2_hbm, src_hbm, dst_hbm, t_hbm, hp_out,
                  rs_out, rows0, rows1, rows2, s1g0, s1g1, s1g2,
                  s2g0, s2g1, s2g2, srcm0, srcm1, srcm2, dstm0, dstm1, dstm2,
                  tm0, tm1, tm2, ss0, ss1, ss2, rs_local, idx_id,
                  hp_acc, rs_shared, lsem0, lsem1, lsem2,
                  gsem0, gsem1, gsem2, ssem0, ssem1, ssem2):
    cid = lax.axis_index("c")
    sid = lax.axis_index("s")
    wid = cid * NS + sid
    base = wid * CPW * K

    rows = [rows0, rows1, rows2]
    s1g = [s1g0, s1g1, s1g2]
    s2g = [s2g0, s2g1, s2g2]
    srcm = [srcm0, srcm1, srcm2]
    dstm = [dstm0, dstm1, dstm2]
    tm = [tm0, tm1, tm2]
    ssrc = [ss0, ss1, ss2]
    lsem = [lsem0, lsem1, lsem2]
    gsem = [gsem0, gsem1, gsem2]
    ssem = [ssem0, ssem1, ssem2]

    zvec = jnp.zeros((L,), jnp.float32)

    def _zero_rows(i, _):
        for j in range(D // L):
            rs_local[i, pl.ds(j * L, L)] = zvec
            rows0[i, pl.ds(j * L, L)] = zvec
        return ()

    lax.fori_loop(0, NH, _zero_rows, ())
    for kblk in range(RPT // K):
        pltpu.sync_copy(rows0, hp_acc.at[pl.ds(sid * RPT + kblk * K, K)])

    @pl.when(sid == 0)
    def _zero_rs_shared():
        pltpu.sync_copy(rows0, rs_shared)

    for g in range(NH // L):
        idx_id[pl.ds(g * L, L)] = lax.iota(jnp.int32, L) + g * L
    plsc.subcore_barrier()

    lanes = lax.iota(jnp.int32, L)
    lane_masks = [lanes == i for i in range(L)]
    bcast_dn = lax.GatherDimensionNumbers(
        offset_dims=(), collapsed_slice_dims=(0,), start_index_map=(0,))

    def _bcast_lane(vec, i):
        idx = jnp.full((L, 1), i, dtype=jnp.int32)
        return lax.gather(vec, idx, bcast_dn, (1,),
                          mode=lax.GatherScatterMode.PROMISE_IN_BOUNDS)

    def _issue_meta(c, b):
        e0 = base + c * K
        pltpu.async_copy(src_hbm.at[pl.ds(e0, K)], srcm[b], lsem[b])
        pltpu.async_copy(dst_hbm.at[pl.ds(e0, K)], dstm[b], lsem[b])
        pltpu.async_copy(t_hbm.at[pl.ds(e0, K)], tm[b], lsem[b])

    def _wait_meta(b):
        pltpu.make_async_copy(src_hbm.at[pl.ds(0, K)], srcm[b], lsem[b]).wait()
        pltpu.make_async_copy(dst_hbm.at[pl.ds(0, K)], dstm[b], lsem[b]).wait()
        pltpu.make_async_copy(t_hbm.at[pl.ds(0, K)], tm[b], lsem[b]).wait()

    def _issue_gather(b):
        pltpu.async_copy(h_hbm.at[dstm[b]], rows[b], gsem[b])
        pltpu.async_copy(s1_hbm.at[srcm[b]], s1g[b], gsem[b])
        pltpu.async_copy(s2_hbm.at[dstm[b]], s2g[b], gsem[b])

    def _wait_gather(b):
        pltpu.make_async_copy(h_hbm.at[dstm[b]], rows[b], gsem[b]).wait()
        pltpu.make_async_copy(s1_hbm.at[srcm[b]], s1g[b], gsem[b]).wait()
        pltpu.make_async_copy(s2_hbm.at[dstm[b]], s2g[b], gsem[b]).wait()

    def _drain_scatter(b):
        pltpu.make_async_copy(rows[b], hp_acc.at[ssrc[b]], ssem[b]).wait()

    _issue_meta(0, 0)
    _issue_meta(1, 1)
    _issue_meta(2, 2)
    _wait_meta(0)
    _issue_gather(0)

    def _triple(q, _):
        for b in range(3):
            c = q * 3 + b
            nb = (b + 1) % 3

            @pl.when(c < CPW)
            def _one():
                _wait_gather(b)

                @pl.when(c + 1 < CPW)
                def _next_gather():
                    _wait_meta(nb)

                    @pl.when(c >= 2)
                    def _free_rows():
                        _drain_scatter(nb)

                    _issue_gather(nb)

                for g in range(K // L):
                    sl = pl.ds(g * L, L)
                    si = srcm[b][sl]
                    score = s1g[b][sl] + s2g[b][sl] + tm[b][sl]
                    e = jnp.exp(-jnp.maximum(score, ALPHA * score))
                    ssrc[b][sl] = si
                    hi = lax.shift_right_logical(si, 7)
                    lo = lax.bitwise_and(si, 127)
                    for i in range(L):
                        row = g * L + i
                        plsc.addupdate_scatter(rs_local, [hi, lo], e,
                                               mask=lane_masks[i])
                        eb = _bcast_lane(e, i)
                        for f in range(D // L):
                            fs = pl.ds(f * L, L)
                            rows[b][row, fs] = rows[b][row, fs] * eb
                pltpu.async_copy(rows[b], hp_acc.at[ssrc[b]], ssem[b],
                                 add=True)

                @pl.when(c + 3 < CPW)
                def _prefetch_meta():
                    _issue_meta(c + 3, b)
        return ()

    lax.fori_loop(0, (CPW + 3) // 3, _triple, ())

    _drain_scatter(2)
    _drain_scatter(0)
    _drain_scatter(1)

    plsc.subcore_barrier()
    pltpu.sync_copy(rs_local, rs_shared.at[idx_id], add=True)
    plsc.subcore_barrier()

    @pl.when(sid == 0)
    def _dump_rs():
        pltpu.sync_copy(rs_shared, rs_out.at[cid])

    for kblk in range(RPT // ZR):
        r0 = sid * RPT + kblk * ZR
        pltpu.sync_copy(hp_acc.at[pl.ds(r0, ZR)], hp_out.at[cid, pl.ds(r0, ZR)])
    plsc.subcore_barrier()


_sc_edge = functools.partial(
    pl.kernel,
    out_type=[
        jax.ShapeDtypeStruct((NC, NPAD, D), jnp.float32),
        jax.ShapeDtypeStruct((NC, NH, D), jnp.float32),
    ],
    mesh=plsc.VectorSubcoreMesh(core_axis_name="c", subcore_axis_name="s",
                                num_cores=NC, num_subcores=NS),
    scratch_types=(
        [pltpu.VMEM((K, D), jnp.float32)] * 3
        + [pltpu.VMEM((K,), jnp.float32)] * 3
        + [pltpu.VMEM((K,), jnp.float32)] * 3
        + [pltpu.VMEM((K,), jnp.int32)] * 3
        + [pltpu.VMEM((K,), jnp.int32)] * 3
        + [pltpu.VMEM((K,), jnp.float32)] * 3
        + [pltpu.VMEM((K,), jnp.int32)] * 3
        + [
            pltpu.VMEM((NH, D), jnp.float32),
            pltpu.VMEM((NH,), jnp.int32),
            pltpu.VMEM_SHARED((NPAD, D), jnp.float32),
            pltpu.VMEM_SHARED((NH, D), jnp.float32),
        ]
        + [pltpu.SemaphoreType.DMA] * 9
    ),
    compiler_params=pltpu.CompilerParams(needs_layout_passes=False),
)(_sc_edge_body)


def _finalize_body(hp_ref, rs_ref, o_ref):
    top = hp_ref[0, :, :] + hp_ref[1, :, :]
    rs = rs_ref[0, :, :] + rs_ref[1, :, :]
    x = top / (rs + 9e-15)
    o_ref[...] = jnp.where(x > 0, x, jnp.exp(jnp.minimum(x, 0.0)) - 1.0)


def _finalize(hp, rs):
    NB = 2000
    return pl.pallas_call(
        _finalize_body,
        grid=(N // NB,),
        in_specs=[
            pl.BlockSpec((NC, NB, D), lambda i: (0, i, 0)),
            pl.BlockSpec((NC, NB, 1), lambda i: (0, i, 0)),
        ],
        out_specs=pl.BlockSpec((NB, D), lambda i: (i, 0)),
        out_shape=jax.ShapeDtypeStruct((N, D), jnp.float32),
    )(hp, rs)


def kernel(input, edge_index, edge_weight_road, edge_r, W, Wr, a):
    h, s1, s2 = _dense_node(input, W, a)
    t = _dense_edge(edge_r.T.reshape(8, ER, D),
                    edge_weight_road.reshape(ER, D), Wr, a)
    src = edge_index[0].astype(jnp.int32)
    dst = edge_index[1].astype(jnp.int32)
    hp, rs = _sc_edge(h, s1.reshape(N), s2.reshape(N), src, dst, t.reshape(E))
    return _finalize(hp, rs.reshape(NC, NH * D)[:, :N].reshape(NC, N, 1))
